# Initial kernel scaffold; baseline (speedup 1.0000x reference)
import jax
import jax.numpy as jnp
from jax.experimental import pallas as pl


def kernel(pos, reflectance, lw1, lb1, lg1, lbe1, lw2, lb2, lg2, lbe2, ew, eb, eg, ebe, d1w, d1b, d1g, d1be, p1w, p1b, p1g, p1be, c1g, c1be, d2w, d2b, d2g, d2be, p2w, p2b, p2g, p2be, c2g, c2be, pw, pb, pg, pbe, edge_src, edge_dst, idx):
    raise NotImplementedError("write your pallas kernel here")


# speedup vs baseline: 1.1238x; 1.1238x over previous
import functools

import jax
import jax.numpy as jnp
from jax import lax
from jax.experimental import pallas as pl
from jax.experimental.pallas import tpu as pltpu
from jax.experimental.pallas import tpu_sc as plsc

_EPS = 1e-5
_NW = 32


def _silu(x):
    return x * jax.nn.sigmoid(x)


def _gather_rel(cols, idx, edge_src, edge_dst):
    E = edge_src.shape[0]
    CE = 128
    n_chunks = E // CE
    rounds = (n_chunks + _NW - 1) // _NW
    mesh = plsc.VectorSubcoreMesh(core_axis_name="c", subcore_axis_name="s")

    @functools.partial(
        pl.kernel,
        mesh=mesh,
        out_type=jax.ShapeDtypeStruct((8 * E,), jnp.float32),
        scratch_types=[
            pltpu.VMEM((CE,), jnp.int32),
            pltpu.VMEM((CE,), jnp.int32),
            pltpu.VMEM((CE,), jnp.int32),
            pltpu.VMEM((4, CE), jnp.float32),
            pltpu.VMEM((4, CE), jnp.float32),
            pltpu.VMEM((CE,), jnp.float32),
            pltpu.SemaphoreType.DMA,
        ],
    )
    def k(cx_hbm, cy_hbm, cz_hbm, cr_hbm, idx_hbm, src_hbm, dst_hbm, out_hbm,
          srcv, dstv, idx2v, scols, dcols, zbuf, sem):
        wid = lax.axis_index("s") * 2 + lax.axis_index("c")
        col_hbm = [cx_hbm, cy_hbm, cz_hbm, cr_hbm]
        zeros16 = jnp.zeros((16,), jnp.float32)
        for t in range(CE // 16):
            zbuf[pl.ds(t * 16, 16)] = zeros16

        def round_body(r, carry):
            c = r * _NW + wid

            @pl.when(c < n_chunks)
            def _():
                base = c * CE
                pltpu.sync_copy(src_hbm.at[pl.ds(base, CE)], srcv)
                pltpu.sync_copy(dst_hbm.at[pl.ds(base, CE)], dstv)
                pltpu.async_copy(idx_hbm.at[dstv], idx2v, sem).wait()
                for cc in range(4):
                    pltpu.async_copy(
                        col_hbm[cc].at[srcv], scols.at[cc], sem).wait()
                    pltpu.async_copy(
                        col_hbm[cc].at[idx2v], dcols.at[cc], sem).wait()
                for cc in range(4):
                    for g in range(CE // 16):
                        sl = pl.ds(g * 16, 16)
                        scols[cc, sl] = scols[cc, sl] - dcols[cc, sl]
                    pltpu.sync_copy(
                        scols.at[cc], out_hbm.at[pl.ds(cc * E + base, CE)])
                    pltpu.sync_copy(
                        zbuf, out_hbm.at[pl.ds((4 + cc) * E + base, CE)])
            return carry

        lax.fori_loop(0, rounds, round_body, 0)

    out = k(cols[0], cols[1], cols[2], cols[3], idx, edge_src, edge_dst)
    return out.reshape(8, E)


def _segment_max(h2pad, edge_dst_pad, vb, m_pad, rm):
    nr = m_pad // rm
    CE = 64
    mesh = plsc.VectorSubcoreMesh(core_axis_name="c", subcore_axis_name="s")
    ranges_per_w = nr // _NW

    @functools.partial(
        pl.kernel,
        mesh=mesh,
        out_type=jax.ShapeDtypeStruct((m_pad, 256), jnp.float32),
        scratch_types=[
            pltpu.VMEM((vb.shape[0], 16), jnp.int32),
            pltpu.VMEM((rm, 256), jnp.float32),
            pltpu.VMEM((CE, 256), jnp.float32),
            pltpu.VMEM((CE,), jnp.int32),
        ],
    )
    def k(h2_hbm, dst_hbm, vb_hbm, out_hbm, vbv, slab, hbuf, dbuf):
        wid = lax.axis_index("s") * 2 + lax.axis_index("c")
        pltpu.sync_copy(vb_hbm, vbv)
        neg_inf = jnp.full((16,), -jnp.inf, dtype=jnp.float32)

        for rr in range(ranges_per_w):
            r = wid * ranges_per_w + rr
            d_base = r * rm
            vbvec = vbv[r, :]
            e0 = vbvec[0]
            e1 = vbvec[1]

            def init_body(j, c2):
                for f in range(16):
                    slab[j, pl.ds(f * 16, 16)] = neg_inf
                return c2

            lax.fori_loop(0, rm, init_body, 0)

            e0a = (e0 // 8) * 8
            n_ch = (e1 - e0a + CE - 1) // CE

            def chunk_body(kk, c3):
                e = pl.multiple_of(e0a + kk * CE, 8)
                pltpu.sync_copy(dst_hbm.at[pl.ds(e, CE)], dbuf)
                pltpu.sync_copy(h2_hbm.at[pl.ds(e, CE), :], hbuf)

                def group_body(g, c2):
                    dvec = dbuf[pl.ds(g * 16, 16)] - d_base
                    for j in range(16):
                        ee = e + g * 16 + j

                        @pl.when(jnp.logical_and(ee >= e0, ee < e1))
                        def _():
                            d = dvec[j]
                            for f in range(16):
                                sl = pl.ds(f * 16, 16)
                                slab[d, sl] = jnp.maximum(
                                    slab[d, sl], hbuf[g * 16 + j, sl])
                    return c2

                lax.fori_loop(0, CE // 16, group_body, 0)
                return c3

            lax.fori_loop(0, n_ch, chunk_body, 0)

            def fin_body(j, c2):
                for f in range(16):
                    sl = pl.ds(f * 16, 16)
                    v = slab[j, sl]
                    slab[j, sl] = jnp.where(v == -jnp.inf, 0.0, v)
                return c2

            lax.fori_loop(0, rm, fin_body, 0)
            pltpu.sync_copy(slab, out_hbm.at[pl.ds(d_base, rm), :])

    return k(h2pad, edge_dst_pad, vb)


def _tc_pass(blocks, consts, f, out_dim, want_y, want_stats, br, out_rows=None,
             t_blocks=()):
    t_blocks = list(t_blocks)
    rows = blocks[0].shape[0] if blocks else t_blocks[0].shape[1]
    grid = rows // br
    consts = [c if c.ndim == 2 else c[None, :] for c in consts]
    in_specs = [pl.BlockSpec((t.shape[0], br), lambda i: (0, i))
                for t in t_blocks]
    in_specs += [pl.BlockSpec((br, b.shape[1]), lambda i: (i, 0)) for b in blocks]
    in_specs += [pl.BlockSpec(c.shape, lambda i: (0, 0)) for c in consts]
    out_shape, out_specs = [], []
    if want_y:
        r_out = rows if out_rows is None else out_rows
        out_shape.append(jax.ShapeDtypeStruct((r_out, out_dim), jnp.float32))
        out_specs.append(pl.BlockSpec((br, out_dim), lambda i: (i, 0)))
    if want_stats:
        out_shape.append(jax.ShapeDtypeStruct((8, out_dim), jnp.float32))
        out_specs.append(pl.BlockSpec((8, out_dim), lambda i: (0, 0)))
    nb, nc = len(t_blocks) + len(blocks), len(consts)

    def kern(*refs):
        irefs = refs[:nb]
        crefs = refs[nb:nb + nc]
        orefs = refs[nb + nc:]
        y = f(*[x[...] for x in irefs], *[c[...] for c in crefs])
        j = 0
        if want_y:
            orefs[j][...] = y
            j += 1
        if want_stats:
            s = jnp.concatenate(
                [jnp.sum(y, axis=0, keepdims=True),
                 jnp.sum(y * y, axis=0, keepdims=True),
                 jnp.zeros((6, out_dim), jnp.float32)], axis=0)

            @pl.when(pl.program_id(0) == 0)
            def _():
                orefs[j][...] = jnp.zeros((8, out_dim), jnp.float32)

            orefs[j][...] += s

    return pl.pallas_call(
        kern, grid=(grid,), in_specs=in_specs, out_specs=out_specs,
        out_shape=out_shape)(*t_blocks, *blocks, *consts)


def _stats(srow, n):
    mean = srow[0] / n
    var = srow[1] / n - mean * mean
    return mean, var


def _fold(mean, var, g, b):
    a = g / jnp.sqrt(var + _EPS)
    return a, b - mean * a


def kernel(pos, reflectance, lw1, lb1, lg1, lbe1, lw2, lb2, lg2, lbe2,
           ew, eb, eg, ebe, d1w, d1b, d1g, d1be, p1w, p1b, p1g, p1be,
           c1g, c1be, d2w, d2b, d2g, d2be, p2w, p2b, p2g, p2be,
           c2g, c2be, pw, pb, pg, pbe, edge_src, edge_dst, idx):
    N = pos.shape[0]
    M = idx.shape[0]
    E = edge_src.shape[0]
    H1 = lw1.shape[1]
    H2 = lw2.shape[1]
    EXP = ew.shape[1]
    BRE = 640
    BRM = 200
    RM = 392
    M_PAD = 25088
    NR = M_PAD // RM

    lw1p = jnp.pad(lw1, ((0, 4), (0, 0)))
    cols = [pos[:, 0], pos[:, 1], pos[:, 2], reflectance]
    rel8 = _gather_rel(cols, idx, edge_src, edge_dst)

    def _dgT(t, w):
        return lax.dot_general(t, w, (((0,), (0,)), ((), ())),
                               preferred_element_type=jnp.float32)

    (st1,) = _tc_pass(
        [], [lw1p, lb1], lambda t, w, b: _dgT(t, w) + b,
        H1, False, True, BRE, t_blocks=[rel8])
    a1, b1 = _fold(*_stats(st1, E), lg1, lbe1)

    def f_y2(t, w1, bb1, ca1, cb1, w2, bb2):
        y1 = _dgT(t, w1) + bb1
        h = _silu(y1 * ca1 + cb1)
        return jnp.dot(h, w2, preferred_element_type=jnp.float32) + bb2

    (st2,) = _tc_pass([], [lw1p, lb1, a1, b1, lw2, lb2], f_y2,
                      H2, False, True, BRE, t_blocks=[rel8])
    a2, b2 = _fold(*_stats(st2, E), lg2, lbe2)

    def f_h2(t, w1, bb1, ca1, cb1, w2, bb2, ca2, cb2):
        y2 = f_y2(t, w1, bb1, ca1, cb1, w2, bb2)
        return _silu(y2 * ca2 + cb2)

    (h2pad,) = _tc_pass([], [lw1p, lb1, a1, b1, lw2, lb2, a2, b2], f_h2,
                        H2, True, False, BRE, out_rows=E + 64,
                        t_blocks=[rel8])

    vb = jnp.searchsorted(
        edge_dst, (jnp.arange(NR + 1) * RM).astype(jnp.int32)).astype(jnp.int32)
    vbt = (jnp.zeros((NR, 16), jnp.int32)
           .at[:, 0].set(vb[:NR]).at[:, 1].set(vb[1:NR + 1]))
    edge_dst_pad = jnp.pad(edge_dst, (0, 64))
    xpad = _segment_max(h2pad, edge_dst_pad, vbt, M_PAD, RM)
    x = xpad[:M]

    dot = lambda t, w: jnp.dot(t, w, preferred_element_type=jnp.float32)
    a0, sA0 = _tc_pass([x], [ew, eb], lambda t, w, b: dot(t, w) + b,
                       EXP, True, True, BRM)
    ca0, cb0 = _fold(*_stats(sA0, M), eg, ebe)

    (sT0,) = _tc_pass([a0], [ca0, cb0], lambda t, a, b: _silu(t * a + b),
                      EXP, False, True, BRM)
    mT0, vT0 = _stats(sT0, M)
    ca1n, cb1n = _fold(mT0 * d1w + d1b, vT0 * d1w * d1w, d1g, d1be)

    def f_p3(t, a_, b_, dw, db, a1_, b1_, w, b):
        t0 = _silu(t * a_ + b_)
        t1 = _silu((t0 * dw + db) * a1_ + b1_)
        return dot(t1, w) + b

    a2n, sA2 = _tc_pass([a0], [ca0, cb0, d1w, d1b, ca1n, cb1n, p1w, p1b],
                        f_p3, EXP, True, True, BRM)
    ca2n, cb2n = _fold(*_stats(sA2, M), p1g, p1be)

    (sT2,) = _tc_pass([a2n], [ca2n, cb2n], lambda t, a, b: _silu(t * a + b),
                      EXP, False, True, BRM)
    cc1a, cc1b = _fold(*_stats(sT2, M), c1g, c1be)

    (sT3,) = _tc_pass(
        [a2n], [ca2n, cb2n, cc1a, cc1b],
        lambda t, a, b, a3, b3: _silu(_silu(t * a + b) * a3 + b3),
        EXP, False, True, BRM)
    mT3, vT3 = _stats(sT3, M)
    ca4, cb4 = _fold(mT3 * d2w + d2b, vT3 * d2w * d2w, d2g, d2be)

    def f_p6(t, a_, b_, a3, b3, dw, db, a4_, b4_, w, b):
        t2 = _silu(t * a_ + b_)
        t3 = _silu(t2 * a3 + b3)
        t4 = _silu((t3 * dw + db) * a4_ + b4_)
        return dot(t4, w) + b

    a5, sA5 = _tc_pass(
        [a2n], [ca2n, cb2n, cc1a, cc1b, d2w, d2b, ca4, cb4, p2w, p2b],
        f_p6, EXP, True, True, BRM)
    ca5, cb5 = _fold(*_stats(sA5, M), p2g, p2be)

    (sT5,) = _tc_pass([a5], [ca5, cb5], lambda t, a, b: _silu(t * a + b),
                      EXP, False, True, BRM)
    cc2a, cc2b = _fold(*_stats(sT5, M), c2g, c2be)

    def f_p8(t, a_, b_, a6, b6, w, b):
        t5 = _silu(t * a_ + b_)
        t6 = t5 * a6 + b6
        return dot(t6, w) + b

    a7, sA7 = _tc_pass([a5], [ca5, cb5, cc2a, cc2b, pw, pb], f_p8,
                       H2, True, True, BRM)
    ca7, cb7 = _fold(*_stats(sA7, M), pg, pbe)

    (out,) = _tc_pass([a7, x], [ca7, cb7],
                      lambda t, r, a, b: _silu(t * a + b + r),
                      H2, True, False, BRM)
    return out


# speedup vs baseline: 1.7474x; 1.5548x over previous
import functools

import jax
import jax.numpy as jnp
from jax import lax
from jax.experimental import pallas as pl
from jax.experimental.pallas import tpu as pltpu
from jax.experimental.pallas import tpu_sc as plsc

_EPS = 1e-5
_NW = 32


def _silu(x):
    return x * jax.nn.sigmoid(x)


def _gather_rel(cols, idx, edge_src, edge_dst):
    E = edge_src.shape[0]
    CE = 128
    n_chunks = E // CE
    rounds = (n_chunks + _NW - 1) // _NW
    mesh = plsc.VectorSubcoreMesh(core_axis_name="c", subcore_axis_name="s")

    @functools.partial(
        pl.kernel,
        mesh=mesh,
        out_type=jax.ShapeDtypeStruct((8 * E,), jnp.float32),
        scratch_types=[
            pltpu.VMEM((CE,), jnp.int32),
            pltpu.VMEM((CE,), jnp.int32),
            pltpu.VMEM((CE,), jnp.int32),
            pltpu.VMEM((4, CE), jnp.float32),
            pltpu.VMEM((4, CE), jnp.float32),
            pltpu.VMEM((CE,), jnp.float32),
            pltpu.SemaphoreType.DMA,
        ],
    )
    def k(cx_hbm, cy_hbm, cz_hbm, cr_hbm, idx_hbm, src_hbm, dst_hbm, out_hbm,
          srcv, dstv, idx2v, scols, dcols, zbuf, sem):
        wid = lax.axis_index("s") * 2 + lax.axis_index("c")
        col_hbm = [cx_hbm, cy_hbm, cz_hbm, cr_hbm]
        zeros16 = jnp.zeros((16,), jnp.float32)
        for t in range(CE // 16):
            zbuf[pl.ds(t * 16, 16)] = zeros16

        def round_body(r, carry):
            c = r * _NW + wid

            @pl.when(c < n_chunks)
            def _():
                base = c * CE
                pltpu.sync_copy(src_hbm.at[pl.ds(base, CE)], srcv)
                pltpu.sync_copy(dst_hbm.at[pl.ds(base, CE)], dstv)
                pltpu.async_copy(idx_hbm.at[dstv], idx2v, sem).wait()
                cps = [pltpu.async_copy(col_hbm[cc].at[srcv], scols.at[cc],
                                        sem) for cc in range(4)]
                cps += [pltpu.async_copy(col_hbm[cc].at[idx2v], dcols.at[cc],
                                         sem) for cc in range(4)]
                for cp in cps:
                    cp.wait()
                for cc in range(4):
                    for g in range(CE // 16):
                        sl = pl.ds(g * 16, 16)
                        scols[cc, sl] = scols[cc, sl] - dcols[cc, sl]
                    pltpu.sync_copy(
                        scols.at[cc], out_hbm.at[pl.ds(cc * E + base, CE)])
                    pltpu.sync_copy(
                        zbuf, out_hbm.at[pl.ds((4 + cc) * E + base, CE)])
            return carry

        lax.fori_loop(0, rounds, round_body, 0)

    out = k(cols[0], cols[1], cols[2], cols[3], idx, edge_src, edge_dst)
    return out.reshape(8, E)


def _segment_max(h2pad, edge_dst_pad, vb, m_pad, rm):
    nr = m_pad // rm
    CE = 64
    mesh = plsc.VectorSubcoreMesh(core_axis_name="c", subcore_axis_name="s")
    ranges_per_w = nr // _NW

    @functools.partial(
        pl.kernel,
        mesh=mesh,
        out_type=jax.ShapeDtypeStruct((m_pad, 256), jnp.float32),
        scratch_types=[
            pltpu.VMEM((vb.shape[0], 16), jnp.int32),
            pltpu.VMEM((rm, 256), jnp.float32),
            pltpu.VMEM((CE, 256), jnp.float32),
            pltpu.VMEM((CE,), jnp.int32),
        ],
    )
    def k(h2_hbm, dst_hbm, vb_hbm, out_hbm, vbv, slab, hbuf, dbuf):
        wid = lax.axis_index("s") * 2 + lax.axis_index("c")
        pltpu.sync_copy(vb_hbm, vbv)
        neg_inf = jnp.full((16,), -jnp.inf, dtype=jnp.float32)

        for rr in range(ranges_per_w):
            r = wid * ranges_per_w + rr
            d_base = r * rm
            vbvec = vbv[r, :]
            e0 = vbvec[0]
            e1 = vbvec[1]

            def init_body(j, c2):
                for f in range(16):
                    slab[j, pl.ds(f * 16, 16)] = neg_inf
                return c2

            lax.fori_loop(0, rm, init_body, 0)

            e0a = (e0 // 8) * 8
            n_ch = (e1 - e0a + CE - 1) // CE

            def chunk_body(kk, c3):
                e = pl.multiple_of(e0a + kk * CE, 8)
                pltpu.sync_copy(dst_hbm.at[pl.ds(e, CE)], dbuf)
                pltpu.sync_copy(h2_hbm.at[pl.ds(e, CE), :], hbuf)

                def group_body(g, c2):
                    dvec = dbuf[pl.ds(g * 16, 16)] - d_base
                    for j in range(16):
                        ee = e + g * 16 + j

                        @pl.when(jnp.logical_and(ee >= e0, ee < e1))
                        def _():
                            d = dvec[j]
                            for f in range(16):
                                sl = pl.ds(f * 16, 16)
                                slab[d, sl] = jnp.maximum(
                                    slab[d, sl], hbuf[g * 16 + j, sl])
                    return c2

                lax.fori_loop(0, CE // 16, group_body, 0)
                return c3

            lax.fori_loop(0, n_ch, chunk_body, 0)

            def fin_body(j, c2):
                for f in range(16):
                    sl = pl.ds(f * 16, 16)
                    v = slab[j, sl]
                    slab[j, sl] = jnp.where(v == -jnp.inf, 0.0, v)
                return c2

            lax.fori_loop(0, rm, fin_body, 0)
            pltpu.sync_copy(slab, out_hbm.at[pl.ds(d_base, rm), :])

    return k(h2pad, edge_dst_pad, vb)


def _tc_pass(blocks, consts, f, out_dim, want_y, want_stats, br, out_rows=None,
             t_blocks=()):
    t_blocks = list(t_blocks)
    rows = blocks[0].shape[0] if blocks else t_blocks[0].shape[1]
    grid = rows // br
    consts = [c if c.ndim == 2 else c[None, :] for c in consts]
    in_specs = [pl.BlockSpec((t.shape[0], br), lambda i: (0, i))
                for t in t_blocks]
    in_specs += [pl.BlockSpec((br, b.shape[1]), lambda i: (i, 0)) for b in blocks]
    in_specs += [pl.BlockSpec(c.shape, lambda i: (0, 0)) for c in consts]
    out_shape, out_specs = [], []
    if want_y:
        r_out = rows if out_rows is None else out_rows
        out_shape.append(jax.ShapeDtypeStruct((r_out, out_dim), jnp.float32))
        out_specs.append(pl.BlockSpec((br, out_dim), lambda i: (i, 0)))
    if want_stats:
        out_shape.append(jax.ShapeDtypeStruct((8, out_dim), jnp.float32))
        out_specs.append(pl.BlockSpec((8, out_dim), lambda i: (0, 0)))
    nb, nc = len(t_blocks) + len(blocks), len(consts)

    def kern(*refs):
        irefs = refs[:nb]
        crefs = refs[nb:nb + nc]
        orefs = refs[nb + nc:]
        y = f(*[x[...] for x in irefs], *[c[...] for c in crefs])
        j = 0
        if want_y:
            orefs[j][...] = y
            j += 1
        if want_stats:
            s = jnp.concatenate(
                [jnp.sum(y, axis=0, keepdims=True),
                 jnp.sum(y * y, axis=0, keepdims=True),
                 jnp.zeros((6, out_dim), jnp.float32)], axis=0)

            @pl.when(pl.program_id(0) == 0)
            def _():
                orefs[j][...] = jnp.zeros((8, out_dim), jnp.float32)

            orefs[j][...] += s

    return pl.pallas_call(
        kern, grid=(grid,), in_specs=in_specs, out_specs=out_specs,
        out_shape=out_shape)(*t_blocks, *blocks, *consts)


def _stats(srow, n):
    mean = srow[0] / n
    var = srow[1] / n - mean * mean
    return mean, var


def _fold(mean, var, g, b):
    a = g / jnp.sqrt(var + _EPS)
    return a, b - mean * a


def kernel(pos, reflectance, lw1, lb1, lg1, lbe1, lw2, lb2, lg2, lbe2,
           ew, eb, eg, ebe, d1w, d1b, d1g, d1be, p1w, p1b, p1g, p1be,
           c1g, c1be, d2w, d2b, d2g, d2be, p2w, p2b, p2g, p2be,
           c2g, c2be, pw, pb, pg, pbe, edge_src, edge_dst, idx):
    N = pos.shape[0]
    M = idx.shape[0]
    E = edge_src.shape[0]
    H1 = lw1.shape[1]
    H2 = lw2.shape[1]
    EXP = ew.shape[1]
    BRE = 3200
    BRM = 1000
    RM = 392
    M_PAD = 25088
    NR = M_PAD // RM

    lw1p = jnp.pad(lw1, ((0, 4), (0, 0)))
    cols = [pos[:, 0], pos[:, 1], pos[:, 2], reflectance]
    rel8 = _gather_rel(cols, idx, edge_src, edge_dst)

    def _dgT(t, w):
        return lax.dot_general(t, w, (((0,), (0,)), ((), ())),
                               preferred_element_type=jnp.float32)

    (st1,) = _tc_pass(
        [], [lw1p, lb1], lambda t, w, b: _dgT(t, w) + b,
        H1, False, True, BRE, t_blocks=[rel8])
    a1, b1 = _fold(*_stats(st1, E), lg1, lbe1)

    def f_y2(t, w1, bb1, ca1, cb1, w2, bb2):
        y1 = _dgT(t, w1) + bb1
        h = _silu(y1 * ca1 + cb1)
        return jnp.dot(h, w2, preferred_element_type=jnp.float32) + bb2

    (st2,) = _tc_pass([], [lw1p, lb1, a1, b1, lw2, lb2], f_y2,
                      H2, False, True, BRE, t_blocks=[rel8])
    a2, b2 = _fold(*_stats(st2, E), lg2, lbe2)

    def f_h2(t, w1, bb1, ca1, cb1, w2, bb2, ca2, cb2):
        y2 = f_y2(t, w1, bb1, ca1, cb1, w2, bb2)
        return _silu(y2 * ca2 + cb2)

    (h2pad,) = _tc_pass([], [lw1p, lb1, a1, b1, lw2, lb2, a2, b2], f_h2,
                        H2, True, False, BRE, out_rows=E + 64,
                        t_blocks=[rel8])

    vb = jnp.searchsorted(
        edge_dst, (jnp.arange(NR + 1) * RM).astype(jnp.int32)).astype(jnp.int32)
    vbt = (jnp.zeros((NR, 16), jnp.int32)
           .at[:, 0].set(vb[:NR]).at[:, 1].set(vb[1:NR + 1]))
    edge_dst_pad = jnp.pad(edge_dst, (0, 64))
    xpad = _segment_max(h2pad, edge_dst_pad, vbt, M_PAD, RM)
    x = xpad[:M]

    dot = lambda t, w: jnp.dot(t, w, preferred_element_type=jnp.float32)
    a0, sA0 = _tc_pass([x], [ew, eb], lambda t, w, b: dot(t, w) + b,
                       EXP, True, True, BRM)
    ca0, cb0 = _fold(*_stats(sA0, M), eg, ebe)

    (sT0,) = _tc_pass([a0], [ca0, cb0], lambda t, a, b: _silu(t * a + b),
                      EXP, False, True, BRM)
    mT0, vT0 = _stats(sT0, M)
    ca1n, cb1n = _fold(mT0 * d1w + d1b, vT0 * d1w * d1w, d1g, d1be)

    def f_p3(t, a_, b_, dw, db, a1_, b1_, w, b):
        t0 = _silu(t * a_ + b_)
        t1 = _silu((t0 * dw + db) * a1_ + b1_)
        return dot(t1, w) + b

    a2n, sA2 = _tc_pass([a0], [ca0, cb0, d1w, d1b, ca1n, cb1n, p1w, p1b],
                        f_p3, EXP, True, True, BRM)
    ca2n, cb2n = _fold(*_stats(sA2, M), p1g, p1be)

    (sT2,) = _tc_pass([a2n], [ca2n, cb2n], lambda t, a, b: _silu(t * a + b),
                      EXP, False, True, BRM)
    cc1a, cc1b = _fold(*_stats(sT2, M), c1g, c1be)

    (sT3,) = _tc_pass(
        [a2n], [ca2n, cb2n, cc1a, cc1b],
        lambda t, a, b, a3, b3: _silu(_silu(t * a + b) * a3 + b3),
        EXP, False, True, BRM)
    mT3, vT3 = _stats(sT3, M)
    ca4, cb4 = _fold(mT3 * d2w + d2b, vT3 * d2w * d2w, d2g, d2be)

    def f_p6(t, a_, b_, a3, b3, dw, db, a4_, b4_, w, b):
        t2 = _silu(t * a_ + b_)
        t3 = _silu(t2 * a3 + b3)
        t4 = _silu((t3 * dw + db) * a4_ + b4_)
        return dot(t4, w) + b

    a5, sA5 = _tc_pass(
        [a2n], [ca2n, cb2n, cc1a, cc1b, d2w, d2b, ca4, cb4, p2w, p2b],
        f_p6, EXP, True, True, BRM)
    ca5, cb5 = _fold(*_stats(sA5, M), p2g, p2be)

    (sT5,) = _tc_pass([a5], [ca5, cb5], lambda t, a, b: _silu(t * a + b),
                      EXP, False, True, BRM)
    cc2a, cc2b = _fold(*_stats(sT5, M), c2g, c2be)

    def f_p8(t, a_, b_, a6, b6, w, b):
        t5 = _silu(t * a_ + b_)
        t6 = t5 * a6 + b6
        return dot(t6, w) + b

    a7, sA7 = _tc_pass([a5], [ca5, cb5, cc2a, cc2b, pw, pb], f_p8,
                       H2, True, True, BRM)
    ca7, cb7 = _fold(*_stats(sA7, M), pg, pbe)

    (out,) = _tc_pass([a7, x], [ca7, cb7],
                      lambda t, r, a, b: _silu(t * a + b + r),
                      H2, True, False, BRM)
    return out


# speedup vs baseline: 1.8242x; 1.0440x over previous
import functools

import jax
import jax.numpy as jnp
from jax import lax
from jax.experimental import pallas as pl
from jax.experimental.pallas import tpu as pltpu
from jax.experimental.pallas import tpu_sc as plsc

_EPS = 1e-5
_NW = 32


def _silu(x):
    return x * jax.nn.sigmoid(x)


def _gather_rel(cols, idx, edge_src, edge_dst):
    E = edge_src.shape[0]
    CE = 128
    n_chunks = E // CE
    rounds = (n_chunks + _NW - 1) // _NW
    mesh = plsc.VectorSubcoreMesh(core_axis_name="c", subcore_axis_name="s")

    @functools.partial(
        pl.kernel,
        mesh=mesh,
        out_type=jax.ShapeDtypeStruct((8 * E,), jnp.float32),
        scratch_types=[
            pltpu.VMEM((CE,), jnp.int32),
            pltpu.VMEM((CE,), jnp.int32),
            pltpu.VMEM((CE,), jnp.int32),
            pltpu.VMEM((4, CE), jnp.float32),
            pltpu.VMEM((4, CE), jnp.float32),
            pltpu.VMEM((CE,), jnp.float32),
            pltpu.SemaphoreType.DMA,
        ],
    )
    def k(cx_hbm, cy_hbm, cz_hbm, cr_hbm, idx_hbm, src_hbm, dst_hbm, out_hbm,
          srcv, dstv, idx2v, scols, dcols, zbuf, sem):
        wid = lax.axis_index("s") * 2 + lax.axis_index("c")
        col_hbm = [cx_hbm, cy_hbm, cz_hbm, cr_hbm]
        zeros16 = jnp.zeros((16,), jnp.float32)
        for t in range(CE // 16):
            zbuf[pl.ds(t * 16, 16)] = zeros16

        def round_body(r, carry):
            c = r * _NW + wid

            @pl.when(c < n_chunks)
            def _():
                base = c * CE
                pltpu.sync_copy(src_hbm.at[pl.ds(base, CE)], srcv)
                pltpu.sync_copy(dst_hbm.at[pl.ds(base, CE)], dstv)
                pltpu.async_copy(idx_hbm.at[dstv], idx2v, sem).wait()
                cps = [pltpu.async_copy(col_hbm[cc].at[srcv], scols.at[cc],
                                        sem) for cc in range(4)]
                cps += [pltpu.async_copy(col_hbm[cc].at[idx2v], dcols.at[cc],
                                         sem) for cc in range(4)]
                for cp in cps:
                    cp.wait()
                for cc in range(4):
                    for g in range(CE // 16):
                        sl = pl.ds(g * 16, 16)
                        scols[cc, sl] = scols[cc, sl] - dcols[cc, sl]
                    pltpu.sync_copy(
                        scols.at[cc], out_hbm.at[pl.ds(cc * E + base, CE)])
                    pltpu.sync_copy(
                        zbuf, out_hbm.at[pl.ds((4 + cc) * E + base, CE)])
            return carry

        lax.fori_loop(0, rounds, round_body, 0)

    out = k(cols[0], cols[1], cols[2], cols[3], idx, edge_src, edge_dst)
    return out.reshape(8, E)


def _segment_max(h2pad, edge_dst_pad, vb, m_pad, rm):
    nr = m_pad // rm
    CE = 128
    mesh = plsc.VectorSubcoreMesh(core_axis_name="c", subcore_axis_name="s")
    ranges_per_w = nr // _NW

    @functools.partial(
        pl.kernel,
        mesh=mesh,
        out_type=jax.ShapeDtypeStruct((m_pad, 256), jnp.float32),
        scratch_types=[
            pltpu.VMEM((vb.shape[0], 16), jnp.int32),
            pltpu.VMEM((rm, 256), jnp.float32),
            pltpu.VMEM((CE, 256), jnp.float32),
            pltpu.VMEM((CE,), jnp.int32),
            pltpu.SemaphoreType.DMA,
        ],
    )
    def k(h2_hbm, dst_hbm, vb_hbm, out_hbm, vbv, slab, hbuf, dbuf, sem):
        wid = lax.axis_index("s") * 2 + lax.axis_index("c")
        pltpu.sync_copy(vb_hbm, vbv)
        neg_inf = jnp.full((16,), -jnp.inf, dtype=jnp.float32)

        for rr in range(ranges_per_w):
            r = wid * ranges_per_w + rr
            d_base = r * rm
            vbvec = vbv[r, :]
            e0 = vbvec[0]
            e1 = vbvec[1]

            def init_body(j, c2):
                for f in range(16):
                    slab[j, pl.ds(f * 16, 16)] = neg_inf
                return c2

            lax.fori_loop(0, rm, init_body, 0)

            e0a = (e0 // 8) * 8
            n_ch = (e1 - e0a + CE - 1) // CE

            def chunk_body(kk, c3):
                e = pl.multiple_of(e0a + kk * CE, 8)
                cp1 = pltpu.async_copy(dst_hbm.at[pl.ds(e, CE)], dbuf, sem)
                cp2 = pltpu.async_copy(h2_hbm.at[pl.ds(e, CE), :], hbuf, sem)
                cp1.wait()
                cp2.wait()

                def group_body(g, c2):
                    dvec = dbuf[pl.ds(g * 16, 16)] - d_base
                    for j in range(16):
                        ee = e + g * 16 + j

                        @pl.when(jnp.logical_and(ee >= e0, ee < e1))
                        def _():
                            d = dvec[j]
                            for f in range(16):
                                sl = pl.ds(f * 16, 16)
                                slab[d, sl] = jnp.maximum(
                                    slab[d, sl], hbuf[g * 16 + j, sl])
                    return c2

                lax.fori_loop(0, CE // 16, group_body, 0)
                return c3

            lax.fori_loop(0, n_ch, chunk_body, 0)

            def fin_body(j, c2):
                for f in range(16):
                    sl = pl.ds(f * 16, 16)
                    v = slab[j, sl]
                    slab[j, sl] = jnp.where(v == -jnp.inf, 0.0, v)
                return c2

            lax.fori_loop(0, rm, fin_body, 0)
            pltpu.sync_copy(slab, out_hbm.at[pl.ds(d_base, rm), :])

    return k(h2pad, edge_dst_pad, vb)


def _tc_pass(blocks, consts, f, out_dim, want_y, want_stats, br, out_rows=None,
             t_blocks=()):
    t_blocks = list(t_blocks)
    rows = blocks[0].shape[0] if blocks else t_blocks[0].shape[1]
    grid = rows // br
    consts = [c if c.ndim == 2 else c[None, :] for c in consts]
    in_specs = [pl.BlockSpec((t.shape[0], br), lambda i: (0, i))
                for t in t_blocks]
    in_specs += [pl.BlockSpec((br, b.shape[1]), lambda i: (i, 0)) for b in blocks]
    in_specs += [pl.BlockSpec(c.shape, lambda i: (0, 0)) for c in consts]
    out_shape, out_specs = [], []
    if want_y:
        r_out = rows if out_rows is None else out_rows
        out_shape.append(jax.ShapeDtypeStruct((r_out, out_dim), jnp.float32))
        out_specs.append(pl.BlockSpec((br, out_dim), lambda i: (i, 0)))
    if want_stats:
        out_shape.append(jax.ShapeDtypeStruct((8, out_dim), jnp.float32))
        out_specs.append(pl.BlockSpec((8, out_dim), lambda i: (0, 0)))
    nb, nc = len(t_blocks) + len(blocks), len(consts)

    def kern(*refs):
        irefs = refs[:nb]
        crefs = refs[nb:nb + nc]
        orefs = refs[nb + nc:]
        y = f(*[x[...] for x in irefs], *[c[...] for c in crefs])
        j = 0
        if want_y:
            orefs[j][...] = y
            j += 1
        if want_stats:
            s = jnp.concatenate(
                [jnp.sum(y, axis=0, keepdims=True),
                 jnp.sum(y * y, axis=0, keepdims=True),
                 jnp.zeros((6, out_dim), jnp.float32)], axis=0)

            @pl.when(pl.program_id(0) == 0)
            def _():
                orefs[j][...] = jnp.zeros((8, out_dim), jnp.float32)

            orefs[j][...] += s

    return pl.pallas_call(
        kern, grid=(grid,), in_specs=in_specs, out_specs=out_specs,
        out_shape=out_shape)(*t_blocks, *blocks, *consts)


def _stats(srow, n):
    mean = srow[0] / n
    var = srow[1] / n - mean * mean
    return mean, var


def _fold(mean, var, g, b):
    a = g / jnp.sqrt(var + _EPS)
    return a, b - mean * a


def kernel(pos, reflectance, lw1, lb1, lg1, lbe1, lw2, lb2, lg2, lbe2,
           ew, eb, eg, ebe, d1w, d1b, d1g, d1be, p1w, p1b, p1g, p1be,
           c1g, c1be, d2w, d2b, d2g, d2be, p2w, p2b, p2g, p2be,
           c2g, c2be, pw, pb, pg, pbe, edge_src, edge_dst, idx):
    N = pos.shape[0]
    M = idx.shape[0]
    E = edge_src.shape[0]
    H1 = lw1.shape[1]
    H2 = lw2.shape[1]
    EXP = ew.shape[1]
    BRE = 3200
    BRM = 1000
    RM = 264
    M_PAD = 25344
    NR = M_PAD // RM

    lw1p = jnp.pad(lw1, ((0, 4), (0, 0)))
    cols = [pos[:, 0], pos[:, 1], pos[:, 2], reflectance]
    rel8 = _gather_rel(cols, idx, edge_src, edge_dst)

    def _dgT(t, w):
        return lax.dot_general(t, w, (((0,), (0,)), ((), ())),
                               preferred_element_type=jnp.float32)

    (st1,) = _tc_pass(
        [], [lw1p, lb1], lambda t, w, b: _dgT(t, w) + b,
        H1, False, True, BRE, t_blocks=[rel8])
    a1, b1 = _fold(*_stats(st1, E), lg1, lbe1)

    def f_y2(t, w1, bb1, ca1, cb1, w2, bb2):
        y1 = _dgT(t, w1) + bb1
        h = _silu(y1 * ca1 + cb1)
        return jnp.dot(h, w2, preferred_element_type=jnp.float32) + bb2

    (st2,) = _tc_pass([], [lw1p, lb1, a1, b1, lw2, lb2], f_y2,
                      H2, False, True, BRE, t_blocks=[rel8])
    a2, b2 = _fold(*_stats(st2, E), lg2, lbe2)

    def f_h2(t, w1, bb1, ca1, cb1, w2, bb2, ca2, cb2):
        y2 = f_y2(t, w1, bb1, ca1, cb1, w2, bb2)
        return _silu(y2 * ca2 + cb2)

    (h2pad,) = _tc_pass([], [lw1p, lb1, a1, b1, lw2, lb2, a2, b2], f_h2,
                        H2, True, False, BRE, out_rows=E + 128,
                        t_blocks=[rel8])

    vb = jnp.searchsorted(
        edge_dst, (jnp.arange(NR + 1) * RM).astype(jnp.int32)).astype(jnp.int32)
    vbt = (jnp.zeros((NR, 16), jnp.int32)
           .at[:, 0].set(vb[:NR]).at[:, 1].set(vb[1:NR + 1]))
    edge_dst_pad = jnp.pad(edge_dst, (0, 128))
    xpad = _segment_max(h2pad, edge_dst_pad, vbt, M_PAD, RM)
    x = xpad[:M]

    dot = lambda t, w: jnp.dot(t, w, preferred_element_type=jnp.float32)
    a0, sA0 = _tc_pass([x], [ew, eb], lambda t, w, b: dot(t, w) + b,
                       EXP, True, True, BRM)
    ca0, cb0 = _fold(*_stats(sA0, M), eg, ebe)

    (sT0,) = _tc_pass([a0], [ca0, cb0], lambda t, a, b: _silu(t * a + b),
                      EXP, False, True, BRM)
    mT0, vT0 = _stats(sT0, M)
    ca1n, cb1n = _fold(mT0 * d1w + d1b, vT0 * d1w * d1w, d1g, d1be)

    def f_p3(t, a_, b_, dw, db, a1_, b1_, w, b):
        t0 = _silu(t * a_ + b_)
        t1 = _silu((t0 * dw + db) * a1_ + b1_)
        return dot(t1, w) + b

    a2n, sA2 = _tc_pass([a0], [ca0, cb0, d1w, d1b, ca1n, cb1n, p1w, p1b],
                        f_p3, EXP, True, True, BRM)
    ca2n, cb2n = _fold(*_stats(sA2, M), p1g, p1be)

    (sT2,) = _tc_pass([a2n], [ca2n, cb2n], lambda t, a, b: _silu(t * a + b),
                      EXP, False, True, BRM)
    cc1a, cc1b = _fold(*_stats(sT2, M), c1g, c1be)

    (sT3,) = _tc_pass(
        [a2n], [ca2n, cb2n, cc1a, cc1b],
        lambda t, a, b, a3, b3: _silu(_silu(t * a + b) * a3 + b3),
        EXP, False, True, BRM)
    mT3, vT3 = _stats(sT3, M)
    ca4, cb4 = _fold(mT3 * d2w + d2b, vT3 * d2w * d2w, d2g, d2be)

    def f_p6(t, a_, b_, a3, b3, dw, db, a4_, b4_, w, b):
        t2 = _silu(t * a_ + b_)
        t3 = _silu(t2 * a3 + b3)
        t4 = _silu((t3 * dw + db) * a4_ + b4_)
        return dot(t4, w) + b

    a5, sA5 = _tc_pass(
        [a2n], [ca2n, cb2n, cc1a, cc1b, d2w, d2b, ca4, cb4, p2w, p2b],
        f_p6, EXP, True, True, BRM)
    ca5, cb5 = _fold(*_stats(sA5, M), p2g, p2be)

    (sT5,) = _tc_pass([a5], [ca5, cb5], lambda t, a, b: _silu(t * a + b),
                      EXP, False, True, BRM)
    cc2a, cc2b = _fold(*_stats(sT5, M), c2g, c2be)

    def f_p8(t, a_, b_, a6, b6, w, b):
        t5 = _silu(t * a_ + b_)
        t6 = t5 * a6 + b6
        return dot(t6, w) + b

    a7, sA7 = _tc_pass([a5], [ca5, cb5, cc2a, cc2b, pw, pb], f_p8,
                       H2, True, True, BRM)
    ca7, cb7 = _fold(*_stats(sA7, M), pg, pbe)

    (out,) = _tc_pass([a7, x], [ca7, cb7],
                      lambda t, r, a, b: _silu(t * a + b + r),
                      H2, True, False, BRM)
    return out
